# baseline (speedup 1.0000x reference)
import functools

import jax
import jax.numpy as jnp
from jax import lax
from jax.experimental import pallas as pl
from jax.experimental.pallas import tpu as pltpu
from jax.experimental.pallas import tpu_sc as plsc

N = 10000
E = 320000
D_IN = 128
D_H = 64

NC = 2
NS = 16
NW = NC * NS

NPAD = 10240
ROWS_PER_TILE = NPAD // NS
CHUNK = 128
EPT = (E + NW - 1) // NW
EPT = ((EPT + CHUNK - 1) // CHUNK) * CHUNK
NCHUNK_PER_TILE = EPT // CHUNK
E_PAD = EPT * NW

_mesh = plsc.VectorSubcoreMesh(core_axis_name="c", subcore_axis_name="s")
_sc_params = pltpu.CompilerParams(
    needs_layout_passes=False, use_tc_tiling_on_sc=False)


def _hist_body(src_hbm, dst_hbm, degs_hbm,
               sidx_v, didx_v, hs_v, hd_v, tmp_v, accs_v, accd_v,
               sh_s):
    cid = lax.axis_index("c")
    sid = lax.axis_index("s")
    wid = cid * NS + sid
    ebase = pl.multiple_of(wid * EPT, EPT)

    pltpu.sync_copy(src_hbm.at[pl.ds(ebase, EPT)], sidx_v)
    pltpu.sync_copy(dst_hbm.at[pl.ds(ebase, EPT)], didx_v)

    zeros = jnp.zeros((16,), jnp.float32)
    ones = jnp.ones((16,), jnp.float32)

    def zero_body(i, _):
        hs_v[pl.ds(i * 16, 16)] = zeros
        hd_v[pl.ds(i * 16, 16)] = zeros
        return 0
    lax.fori_loop(0, NPAD // 16, zero_body, 0)

    def scat_body(i, _):
        vs = sidx_v[pl.ds(i * 16, 16)]
        vd = didx_v[pl.ds(i * 16, 16)]
        plsc.addupdate_scatter(hs_v, [vs], ones)
        plsc.addupdate_scatter(hd_v, [vd], ones)
        return 0
    lax.fori_loop(0, EPT // 16, scat_body, 0)

    off = pl.multiple_of(sid * ROWS_PER_TILE, ROWS_PER_TILE)

    def reduce_into(h_v, acc_v):
        pltpu.sync_copy(h_v, sh_s.at[sid])
        plsc.subcore_barrier()
        pltpu.sync_copy(sh_s.at[0, pl.ds(off, ROWS_PER_TILE)], acc_v)

        def red_body(t, _):
            pltpu.sync_copy(sh_s.at[t, pl.ds(off, ROWS_PER_TILE)], tmp_v)

            def add_k(k, _):
                sl = pl.ds(k * 16, 16)
                acc_v[sl] = acc_v[sl] + tmp_v[sl]
                return 0
            lax.fori_loop(0, ROWS_PER_TILE // 16, add_k, 0)
            return 0
        lax.fori_loop(1, NS, red_body, 0)
        plsc.subcore_barrier()

    reduce_into(hs_v, accs_v)
    reduce_into(hd_v, accd_v)

    pltpu.sync_copy(accs_v, degs_hbm.at[0, cid, pl.ds(off, ROWS_PER_TILE)])
    pltpu.sync_copy(accd_v, degs_hbm.at[1, cid, pl.ds(off, ROWS_PER_TILE)])


_hist_kernel = pl.kernel(
    _hist_body,
    out_type=jax.ShapeDtypeStruct((2, NC, NPAD), jnp.float32),
    mesh=_mesh,
    scratch_types=[
        pltpu.VMEM((EPT,), jnp.int32),
        pltpu.VMEM((EPT,), jnp.int32),
        pltpu.VMEM((NPAD,), jnp.float32),
        pltpu.VMEM((NPAD,), jnp.float32),
        pltpu.VMEM((ROWS_PER_TILE,), jnp.float32),
        pltpu.VMEM((ROWS_PER_TILE,), jnp.float32),
        pltpu.VMEM((ROWS_PER_TILE,), jnp.float32),
        pltpu.VMEM_SHARED((NS, NPAD), jnp.float32),
    ],
    compiler_params=_sc_params,
)


def _agg_body(h_hbm, z_hbm, srcr_hbm, dstr_hbm, aggp_hbm,
              sidx_v, didx_v, rowbuf_a, sh_agg, sh_h,
              sem_g, sem_s):
    cid = lax.axis_index("c")
    sid = lax.axis_index("s")
    wid = cid * NS + sid
    nbase = pl.multiple_of(sid * ROWS_PER_TILE, ROWS_PER_TILE)

    pltpu.sync_copy(z_hbm.at[pl.ds(nbase, ROWS_PER_TILE)],
                    sh_agg.at[pl.ds(nbase, ROWS_PER_TILE)])
    pltpu.sync_copy(h_hbm.at[pl.ds(nbase, ROWS_PER_TILE)],
                    sh_h.at[pl.ds(nbase, ROWS_PER_TILE)])
    pltpu.sync_copy(srcr_hbm.at[wid], sidx_v)
    pltpu.sync_copy(dstr_hbm.at[wid], didx_v)
    plsc.subcore_barrier()

    def gather(j, b):
        pltpu.async_copy(h_hbm.at[sidx_v.at[j]], rowbuf[b], sem_g[b])

    def wait_gather(j, b):
        pltpu.make_async_copy(h_hbm.at[sidx_v.at[j]], rowbuf[b],
                              sem_g[b]).wait()

    def scatter_sync(j, b):
        pltpu.sync_copy(rowbuf[b], sh_agg.at[didx_v.at[j]], add=True)

    buf4 = rowbuf_a

    def slot(j):
        return pl.multiple_of((j % 3) * CHUNK, CHUNK)

    def buf_at(j):
        return buf4.at[pl.ds(slot(j), CHUNK)]

    pltpu.async_copy(sh_h.at[sidx_v.at[0]], buf_at(0), sem_g.at[0])

    def chunk_body(j, _):
        cur = buf_at(j)
        pltpu.make_async_copy(sh_h.at[sidx_v.at[j]], cur,
                              sem_g.at[j % 2]).wait()

        @pl.when(j >= 2)
        def _():
            pltpu.make_async_copy(buf_at(j - 2), sh_agg.at[didx_v.at[j - 2]],
                                  sem_s.at[j % 2]).wait()

        @pl.when(j < NCHUNK_PER_TILE - 1)
        def _():
            pltpu.async_copy(sh_h.at[sidx_v.at[j + 1]], buf_at(j + 1),
                             sem_g.at[(j + 1) % 2])

        pltpu.async_copy(cur, sh_agg.at[didx_v.at[j]], sem_s.at[j % 2],
                         add=True)
        return 0
    lax.fori_loop(0, NCHUNK_PER_TILE, chunk_body, 0)

    jlast = NCHUNK_PER_TILE - 2
    pltpu.make_async_copy(buf_at(jlast), sh_agg.at[didx_v.at[jlast]],
                          sem_s.at[jlast % 2]).wait()
    pltpu.make_async_copy(buf_at(jlast + 1), sh_agg.at[didx_v.at[jlast + 1]],
                          sem_s.at[(jlast + 1) % 2]).wait()

    plsc.subcore_barrier()
    pltpu.sync_copy(sh_agg.at[pl.ds(nbase, ROWS_PER_TILE)],
                    aggp_hbm.at[cid, pl.ds(nbase, ROWS_PER_TILE)])


_agg_kernel = pl.kernel(
    _agg_body,
    out_type=jax.ShapeDtypeStruct((NC, NPAD, D_H), jnp.float32),
    mesh=_mesh,
    scratch_types=[
        pltpu.VMEM((NCHUNK_PER_TILE, CHUNK), jnp.int32),
        pltpu.VMEM((NCHUNK_PER_TILE, CHUNK), jnp.int32),
        pltpu.VMEM((3 * CHUNK, D_H), jnp.float32),
        pltpu.VMEM_SHARED((NPAD, D_H), jnp.float32),
        pltpu.VMEM_SHARED((NPAD, D_H), jnp.float32),
        pltpu.SemaphoreType.DMA((2,)),
        pltpu.SemaphoreType.DMA((2,)),
    ],
    compiler_params=_sc_params,
)


_BLK = 2048
_GRID = NPAD // _BLK


def _norms(degs_ref):
    d = degs_ref[0] + degs_ref[1]
    return jnp.where(d > 0.0, lax.rsqrt(jnp.maximum(d, 1.0)), 0.0)


def _mm1a_body(x_ref, w_ref, h1_ref):
    h1_ref[...] = jnp.dot(x_ref[...], w_ref[...],
                          preferred_element_type=jnp.float32)


def _mm1a(x_pad, w1):
    return pl.pallas_call(
        _mm1a_body,
        grid=(_GRID,),
        in_specs=[
            pl.BlockSpec((_BLK, D_IN), lambda i: (i, 0)),
            pl.BlockSpec((D_IN, D_H), lambda i: (0, 0)),
        ],
        out_specs=pl.BlockSpec((_BLK, D_H), lambda i: (i, 0)),
        out_shape=jax.ShapeDtypeStruct((NPAD, D_H), jnp.float32),
    )(x_pad, w1)


def _mm1b_body(h1_ref, dsrc_ref, ddst_ref, h1s_ref, nsrc_ref, ndst_ref):
    ns = _norms(dsrc_ref)
    nd = _norms(ddst_ref)
    h1s_ref[...] = h1_ref[...] * ns
    nsrc_ref[...] = ns
    ndst_ref[...] = nd


def _mm1b(h1, dsrc, ddst):
    return pl.pallas_call(
        _mm1b_body,
        grid=(_GRID,),
        in_specs=[
            pl.BlockSpec((_BLK, D_H), lambda i: (i, 0)),
            pl.BlockSpec((2, _BLK, 1), lambda i: (0, i, 0)),
            pl.BlockSpec((2, _BLK, 1), lambda i: (0, i, 0)),
        ],
        out_specs=[
            pl.BlockSpec((_BLK, D_H), lambda i: (i, 0)),
            pl.BlockSpec((_BLK, 1), lambda i: (i, 0)),
            pl.BlockSpec((_BLK, 1), lambda i: (i, 0)),
        ],
        out_shape=[
            jax.ShapeDtypeStruct((NPAD, D_H), jnp.float32),
            jax.ShapeDtypeStruct((NPAD, 1), jnp.float32),
            jax.ShapeDtypeStruct((NPAD, 1), jnp.float32),
        ],
    )(h1, dsrc, ddst)


def _mm2_body(aggp_ref, nsrc_ref, ndst_ref, b_ref, w_ref, h2s_ref):
    agg = aggp_ref[0] + aggp_ref[1]
    b = b_ref[0:1, :]
    z = jnp.maximum(agg * ndst_ref[...] + b, 0.0)
    h2 = jnp.dot(z, w_ref[...], preferred_element_type=jnp.float32)
    h2s_ref[...] = h2 * nsrc_ref[...]


def _mm2(aggp, nsrc, ndst, b1_pad, w2):
    return pl.pallas_call(
        _mm2_body,
        grid=(_GRID,),
        in_specs=[
            pl.BlockSpec((2, _BLK, D_H), lambda i: (0, i, 0)),
            pl.BlockSpec((_BLK, 1), lambda i: (i, 0)),
            pl.BlockSpec((_BLK, 1), lambda i: (i, 0)),
            pl.BlockSpec((8, D_H), lambda i: (0, 0)),
            pl.BlockSpec((D_H, D_H), lambda i: (0, 0)),
        ],
        out_specs=pl.BlockSpec((_BLK, D_H), lambda i: (i, 0)),
        out_shape=jax.ShapeDtypeStruct((NPAD, D_H), jnp.float32),
    )(aggp, nsrc, ndst, b1_pad, w2)


def _final_body(aggp_ref, ndst_ref, b_ref, out_ref):
    agg = aggp_ref[0] + aggp_ref[1]
    b = b_ref[0:1, :]
    out_ref[...] = agg * ndst_ref[...] + b


def _final(aggp, ndst, b2_pad):
    return pl.pallas_call(
        _final_body,
        grid=(_GRID,),
        in_specs=[
            pl.BlockSpec((2, _BLK, D_H), lambda i: (0, i, 0)),
            pl.BlockSpec((_BLK, 1), lambda i: (i, 0)),
            pl.BlockSpec((8, D_H), lambda i: (0, 0)),
        ],
        out_specs=pl.BlockSpec((_BLK, D_H), lambda i: (i, 0)),
        out_shape=jax.ShapeDtypeStruct((NPAD, D_H), jnp.float32),
    )(aggp, ndst, b2_pad)


@jax.jit
def _run(features, edge_index, W1, b1, W2, b2):
    src = edge_index[0]
    dst = edge_index[1]
    pad = jnp.full((E_PAD - E,), N, dtype=jnp.int32)
    src_flat = jnp.concatenate([src, pad])
    dst_flat = jnp.concatenate([dst, pad])
    src_r = src_flat.reshape(NW, NCHUNK_PER_TILE, CHUNK)
    dst_r = dst_flat.reshape(NW, NCHUNK_PER_TILE, CHUNK)

    x_pad = jnp.zeros((NPAD, D_IN), jnp.float32).at[:N].set(features)
    zeros_nd = jnp.zeros((NPAD, D_H), jnp.float32)
    b1_pad = jnp.broadcast_to(b1[None, :], (8, D_H))
    b2_pad = jnp.broadcast_to(b2[None, :], (8, D_H))

    h1 = _mm1a(x_pad, W1)
    degs = _hist_kernel(src_flat, dst_flat)
    dsrc = degs[0].reshape(NC, NPAD, 1)
    ddst = degs[1].reshape(NC, NPAD, 1)

    h1s, nsrc, ndst = _mm1b(h1, dsrc, ddst)
    agg1p = _agg_kernel(h1s, zeros_nd, src_r, dst_r)
    h2s = _mm2(agg1p, nsrc, ndst, b1_pad, W2)
    agg2p = _agg_kernel(h2s, zeros_nd, src_r, dst_r)
    out_full = _final(agg2p, ndst, b2_pad)
    return out_full[:N]


def kernel(features, edge_index, W1, b1, W2, b2):
    return _run(features, edge_index, W1, b1, W2, b2)


# speedup vs baseline: 1.0022x; 1.0022x over previous
import functools

import jax
import jax.numpy as jnp
from jax import lax
from jax.experimental import pallas as pl
from jax.experimental.pallas import tpu as pltpu
from jax.experimental.pallas import tpu_sc as plsc

N = 10000
E = 320000
D_IN = 128
D_H = 64

NC = 2
NS = 16
NW = NC * NS

NPAD = 10240
ROWS_PER_TILE = NPAD // NS
CHUNK = 128
EPT = (E + NW - 1) // NW
EPT = ((EPT + CHUNK - 1) // CHUNK) * CHUNK
NCHUNK_PER_TILE = EPT // CHUNK
E_PAD = EPT * NW

_mesh = plsc.VectorSubcoreMesh(core_axis_name="c", subcore_axis_name="s")
_sc_params = pltpu.CompilerParams(
    needs_layout_passes=False, use_tc_tiling_on_sc=False)


def _hist_body(src_hbm, dst_hbm, degs_hbm,
               sidx_v, didx_v, hs_v, hd_v, tmp_v, accs_v, accd_v,
               sh_s):
    cid = lax.axis_index("c")
    sid = lax.axis_index("s")
    wid = cid * NS + sid
    ebase = pl.multiple_of(wid * EPT, EPT)

    pltpu.sync_copy(src_hbm.at[pl.ds(ebase, EPT)], sidx_v)
    pltpu.sync_copy(dst_hbm.at[pl.ds(ebase, EPT)], didx_v)

    zeros = jnp.zeros((16,), jnp.float32)
    ones = jnp.ones((16,), jnp.float32)

    def zero_body(i, _):
        hs_v[pl.ds(i * 16, 16)] = zeros
        hd_v[pl.ds(i * 16, 16)] = zeros
        return 0
    lax.fori_loop(0, NPAD // 16, zero_body, 0)

    def scat_body(i, _):
        vs = sidx_v[pl.ds(i * 16, 16)]
        vd = didx_v[pl.ds(i * 16, 16)]
        plsc.addupdate_scatter(hs_v, [vs], ones)
        plsc.addupdate_scatter(hd_v, [vd], ones)
        return 0
    lax.fori_loop(0, EPT // 16, scat_body, 0)

    off = pl.multiple_of(sid * ROWS_PER_TILE, ROWS_PER_TILE)

    def reduce_into(h_v, acc_v):
        pltpu.sync_copy(h_v, sh_s.at[sid])
        plsc.subcore_barrier()
        pltpu.sync_copy(sh_s.at[0, pl.ds(off, ROWS_PER_TILE)], acc_v)

        def red_body(t, _):
            pltpu.sync_copy(sh_s.at[t, pl.ds(off, ROWS_PER_TILE)], tmp_v)

            def add_k(k, _):
                sl = pl.ds(k * 16, 16)
                acc_v[sl] = acc_v[sl] + tmp_v[sl]
                return 0
            lax.fori_loop(0, ROWS_PER_TILE // 16, add_k, 0)
            return 0
        lax.fori_loop(1, NS, red_body, 0)
        plsc.subcore_barrier()

    reduce_into(hs_v, accs_v)
    reduce_into(hd_v, accd_v)

    pltpu.sync_copy(accs_v, degs_hbm.at[0, cid, pl.ds(off, ROWS_PER_TILE)])
    pltpu.sync_copy(accd_v, degs_hbm.at[1, cid, pl.ds(off, ROWS_PER_TILE)])


_hist_kernel = pl.kernel(
    _hist_body,
    out_type=jax.ShapeDtypeStruct((2, NC, NPAD), jnp.float32),
    mesh=_mesh,
    scratch_types=[
        pltpu.VMEM((EPT,), jnp.int32),
        pltpu.VMEM((EPT,), jnp.int32),
        pltpu.VMEM((NPAD,), jnp.float32),
        pltpu.VMEM((NPAD,), jnp.float32),
        pltpu.VMEM((ROWS_PER_TILE,), jnp.float32),
        pltpu.VMEM((ROWS_PER_TILE,), jnp.float32),
        pltpu.VMEM((ROWS_PER_TILE,), jnp.float32),
        pltpu.VMEM_SHARED((NS, NPAD), jnp.float32),
    ],
    compiler_params=_sc_params,
)


def _agg_body(h_hbm, z_hbm, srcr_hbm, dstr_hbm, aggp_hbm,
              sidx_v, didx_v, rowbuf_a, sh_agg, sh_h,
              sem_g, sem_s):
    cid = lax.axis_index("c")
    sid = lax.axis_index("s")
    wid = cid * NS + sid
    nbase = pl.multiple_of(sid * ROWS_PER_TILE, ROWS_PER_TILE)

    pltpu.sync_copy(z_hbm.at[pl.ds(nbase, ROWS_PER_TILE)],
                    sh_agg.at[pl.ds(nbase, ROWS_PER_TILE)])
    pltpu.sync_copy(h_hbm.at[pl.ds(nbase, ROWS_PER_TILE)],
                    sh_h.at[pl.ds(nbase, ROWS_PER_TILE)])
    pltpu.sync_copy(srcr_hbm.at[wid], sidx_v)
    pltpu.sync_copy(dstr_hbm.at[wid], didx_v)
    plsc.subcore_barrier()

    def gather(j, b):
        pltpu.async_copy(h_hbm.at[sidx_v.at[j]], rowbuf[b], sem_g[b])

    def wait_gather(j, b):
        pltpu.make_async_copy(h_hbm.at[sidx_v.at[j]], rowbuf[b],
                              sem_g[b]).wait()

    def scatter_sync(j, b):
        pltpu.sync_copy(rowbuf[b], sh_agg.at[didx_v.at[j]], add=True)

    buf4 = rowbuf_a

    def slot(j):
        return pl.multiple_of((j % 3) * CHUNK, CHUNK)

    def buf_at(j):
        return buf4.at[pl.ds(slot(j), CHUNK)]

    pltpu.async_copy(sh_h.at[sidx_v.at[0]], buf_at(0), sem_g.at[0])

    def chunk_body(j, _):
        cur = buf_at(j)
        pltpu.make_async_copy(sh_h.at[sidx_v.at[j]], cur,
                              sem_g.at[j % 2]).wait()

        @pl.when(j >= 2)
        def _():
            pltpu.make_async_copy(buf_at(j - 2), sh_agg.at[didx_v.at[j - 2]],
                                  sem_s.at[j % 2]).wait()

        @pl.when(j < NCHUNK_PER_TILE - 1)
        def _():
            pltpu.async_copy(sh_h.at[sidx_v.at[j + 1]], buf_at(j + 1),
                             sem_g.at[(j + 1) % 2])

        pltpu.async_copy(cur, sh_agg.at[didx_v.at[j]], sem_s.at[j % 2],
                         add=True)
        return 0
    lax.fori_loop(0, NCHUNK_PER_TILE, chunk_body, 0)

    jlast = NCHUNK_PER_TILE - 2
    pltpu.make_async_copy(buf_at(jlast), sh_agg.at[didx_v.at[jlast]],
                          sem_s.at[jlast % 2]).wait()
    pltpu.make_async_copy(buf_at(jlast + 1), sh_agg.at[didx_v.at[jlast + 1]],
                          sem_s.at[(jlast + 1) % 2]).wait()

    plsc.subcore_barrier()
    pltpu.sync_copy(sh_agg.at[pl.ds(nbase, ROWS_PER_TILE)],
                    aggp_hbm.at[cid, pl.ds(nbase, ROWS_PER_TILE)])


_agg_kernel = pl.kernel(
    _agg_body,
    out_type=jax.ShapeDtypeStruct((NC, NPAD, D_H), jnp.float32),
    mesh=_mesh,
    scratch_types=[
        pltpu.VMEM((NCHUNK_PER_TILE, CHUNK), jnp.int32),
        pltpu.VMEM((NCHUNK_PER_TILE, CHUNK), jnp.int32),
        pltpu.VMEM((3 * CHUNK, D_H), jnp.float32),
        pltpu.VMEM_SHARED((NPAD, D_H), jnp.float32),
        pltpu.VMEM_SHARED((NPAD, D_H), jnp.float32),
        pltpu.SemaphoreType.DMA((2,)),
        pltpu.SemaphoreType.DMA((2,)),
    ],
    compiler_params=_sc_params,
)


_BLK = 2048
_GRID = NPAD // _BLK


def _norms(degs_ref):
    d = degs_ref[0] + degs_ref[1]
    return jnp.where(d > 0.0, lax.rsqrt(jnp.maximum(d, 1.0)), 0.0)


def _mm1_body(x_ref, w_ref, dsrc_ref, ddst_ref, h1s_ref, nsrc_ref, ndst_ref):
    ns = _norms(dsrc_ref)
    nd = _norms(ddst_ref)
    h = jnp.dot(x_ref[...], w_ref[...], preferred_element_type=jnp.float32)
    h1s_ref[...] = h * ns
    nsrc_ref[...] = ns
    ndst_ref[...] = nd


def _mm1(x_pad, w1, dsrc, ddst):
    return pl.pallas_call(
        _mm1_body,
        grid=(_GRID,),
        in_specs=[
            pl.BlockSpec((_BLK, D_IN), lambda i: (i, 0)),
            pl.BlockSpec((D_IN, D_H), lambda i: (0, 0)),
            pl.BlockSpec((2, _BLK, 1), lambda i: (0, i, 0)),
            pl.BlockSpec((2, _BLK, 1), lambda i: (0, i, 0)),
        ],
        out_specs=[
            pl.BlockSpec((_BLK, D_H), lambda i: (i, 0)),
            pl.BlockSpec((_BLK, 1), lambda i: (i, 0)),
            pl.BlockSpec((_BLK, 1), lambda i: (i, 0)),
        ],
        out_shape=[
            jax.ShapeDtypeStruct((NPAD, D_H), jnp.float32),
            jax.ShapeDtypeStruct((NPAD, 1), jnp.float32),
            jax.ShapeDtypeStruct((NPAD, 1), jnp.float32),
        ],
    )(x_pad, w1, dsrc, ddst)


def _mm2_body(aggp_ref, nsrc_ref, ndst_ref, b_ref, w_ref, h2s_ref):
    agg = aggp_ref[0] + aggp_ref[1]
    b = b_ref[0:1, :]
    z = jnp.maximum(agg * ndst_ref[...] + b, 0.0)
    h2 = jnp.dot(z, w_ref[...], preferred_element_type=jnp.float32)
    h2s_ref[...] = h2 * nsrc_ref[...]


def _mm2(aggp, nsrc, ndst, b1_pad, w2):
    return pl.pallas_call(
        _mm2_body,
        grid=(_GRID,),
        in_specs=[
            pl.BlockSpec((2, _BLK, D_H), lambda i: (0, i, 0)),
            pl.BlockSpec((_BLK, 1), lambda i: (i, 0)),
            pl.BlockSpec((_BLK, 1), lambda i: (i, 0)),
            pl.BlockSpec((8, D_H), lambda i: (0, 0)),
            pl.BlockSpec((D_H, D_H), lambda i: (0, 0)),
        ],
        out_specs=pl.BlockSpec((_BLK, D_H), lambda i: (i, 0)),
        out_shape=jax.ShapeDtypeStruct((NPAD, D_H), jnp.float32),
    )(aggp, nsrc, ndst, b1_pad, w2)


def _final_body(aggp_ref, ndst_ref, b_ref, out_ref):
    agg = aggp_ref[0] + aggp_ref[1]
    b = b_ref[0:1, :]
    out_ref[...] = agg * ndst_ref[...] + b


def _final(aggp, ndst, b2_pad):
    return pl.pallas_call(
        _final_body,
        grid=(_GRID,),
        in_specs=[
            pl.BlockSpec((2, _BLK, D_H), lambda i: (0, i, 0)),
            pl.BlockSpec((_BLK, 1), lambda i: (i, 0)),
            pl.BlockSpec((8, D_H), lambda i: (0, 0)),
        ],
        out_specs=pl.BlockSpec((_BLK, D_H), lambda i: (i, 0)),
        out_shape=jax.ShapeDtypeStruct((NPAD, D_H), jnp.float32),
    )(aggp, ndst, b2_pad)


@jax.jit
def _run(features, edge_index, W1, b1, W2, b2):
    src = edge_index[0]
    dst = edge_index[1]
    pad = jnp.full((E_PAD - E,), N, dtype=jnp.int32)
    src_flat = jnp.concatenate([src, pad])
    dst_flat = jnp.concatenate([dst, pad])
    src_r = src_flat.reshape(NW, NCHUNK_PER_TILE, CHUNK)
    dst_r = dst_flat.reshape(NW, NCHUNK_PER_TILE, CHUNK)

    x_pad = jnp.zeros((NPAD, D_IN), jnp.float32).at[:N].set(features)
    zeros_nd = jnp.zeros((NPAD, D_H), jnp.float32)
    b1_pad = jnp.broadcast_to(b1[None, :], (8, D_H))
    b2_pad = jnp.broadcast_to(b2[None, :], (8, D_H))

    degs = _hist_kernel(src_flat, dst_flat)
    dsrc = degs[0].reshape(NC, NPAD, 1)
    ddst = degs[1].reshape(NC, NPAD, 1)

    h1s, nsrc, ndst = _mm1(x_pad, W1, dsrc, ddst)
    agg1p = _agg_kernel(h1s, zeros_nd, src_r, dst_r)
    h2s = _mm2(agg1p, nsrc, ndst, b1_pad, W2)
    agg2p = _agg_kernel(h2s, zeros_nd, src_r, dst_r)
    out_full = _final(agg2p, ndst, b2_pad)
    return out_full[:N]


def kernel(features, edge_index, W1, b1, W2, b2):
    return _run(features, edge_index, W1, b1, W2, b2)
